# baseline (speedup 1.0000x reference)
import dataclasses
import functools

import jax
import jax.numpy as jnp
from jax import lax
from jax.experimental import pallas as pl
from jax.experimental.pallas import tpu as pltpu
from jax.experimental.pallas import tpu_sc as plsc

NC = 2
NS = 16
H = 128


def _sc_mesh():
    return plsc.VectorSubcoreMesh(
        core_axis_name="c", subcore_axis_name="s", num_cores=NC, num_subcores=NS)


def _sc_params():
    cp = pltpu.CompilerParams()
    if "needs_layout_passes" in pltpu.CompilerParams.__dataclass_fields__:
        cp = dataclasses.replace(cp, needs_layout_passes=False)
    return cp


def _make_deg_kernel(N, E, C):
    epw = E // (NC * NS)
    nchunks = epw // C
    rpt = N // NS

    assert nchunks % 2 == 1

    @functools.partial(
        pl.kernel,
        out_type=jax.ShapeDtypeStruct((NC, N, H), jnp.float32),
        mesh=_sc_mesh(),
        scratch_types=[
            pltpu.VMEM_SHARED((N, H), jnp.float32),
            pltpu.VMEM((C, H), jnp.float32),
            pltpu.VMEM((C,), jnp.int32),
            pltpu.VMEM((C,), jnp.int32),
            pltpu.SemaphoreType.DMA,
            pltpu.SemaphoreType.DMA,
        ],
    )
    def deg_kernel(dst_hbm, ones_hbm, zeros_hbm, out_hbm, acc_sh, ones_v,
                   idx_a, idx_b, sem_a, sem_b):
        c = lax.axis_index("c")
        s = lax.axis_index("s")
        wid = c * NS + s
        pltpu.sync_copy(zeros_hbm, acc_sh.at[pl.ds(s * rpt, rpt)])
        pltpu.sync_copy(ones_hbm, ones_v)
        plsc.subcore_barrier()
        base = wid * epw

        def start(k, idx_v, sem):
            pltpu.sync_copy(dst_hbm.at[pl.ds(base + k * C, C)], idx_v)
            pltpu.async_copy(ones_v, acc_sh.at[idx_v], sem, add=True)

        def finish(idx_v, sem):
            pltpu.make_async_copy(ones_v, acc_sh.at[idx_v], sem).wait()

        start(0, idx_a, sem_a)

        def body(i, carry):
            start(2 * i + 1, idx_b, sem_b)
            finish(idx_a, sem_a)
            start(2 * i + 2, idx_a, sem_a)
            finish(idx_b, sem_b)
            return carry

        lax.fori_loop(0, (nchunks - 1) // 2, body, 0)
        finish(idx_a, sem_a)
        plsc.subcore_barrier()
        pltpu.sync_copy(acc_sh.at[pl.ds(s * rpt, rpt)],
                        out_hbm.at[c, pl.ds(s * rpt, rpt)])

    return deg_kernel


def _make_scatter_kernel(N, E, C):
    epw = E // (NC * NS)
    nchunks = epw // C
    rpt = N // NS

    assert nchunks % 2 == 1

    @functools.partial(
        pl.kernel,
        out_type=jax.ShapeDtypeStruct((NC, N, H), jnp.float32),
        mesh=_sc_mesh(),
        scratch_types=[
            pltpu.VMEM_SHARED((N, H), jnp.float32),
            pltpu.VMEM((C, H), jnp.float32),
            pltpu.VMEM((C, H), jnp.float32),
            pltpu.VMEM((C,), jnp.int32),
            pltpu.VMEM((C,), jnp.int32),
            pltpu.VMEM((C,), jnp.int32),
            pltpu.VMEM((C,), jnp.int32),
            pltpu.SemaphoreType.DMA,
            pltpu.SemaphoreType.DMA,
        ],
    )
    def scatter_kernel(y_hbm, src_hbm, dst_hbm, zeros_hbm, out_hbm,
                       acc_sh, rows_a, rows_b, isrc_a, isrc_b,
                       idst_a, idst_b, sem_a, sem_b):
        c = lax.axis_index("c")
        s = lax.axis_index("s")
        wid = c * NS + s
        pltpu.sync_copy(zeros_hbm, acc_sh.at[pl.ds(s * rpt, rpt)])
        plsc.subcore_barrier()
        base = wid * epw

        def start(k, isrc, idst, rows, sem):
            pltpu.sync_copy(src_hbm.at[pl.ds(base + k * C, C)], isrc)
            pltpu.sync_copy(dst_hbm.at[pl.ds(base + k * C, C)], idst)
            pltpu.async_copy(y_hbm.at[isrc], rows, sem)

        def finish(isrc, idst, rows, sem):
            pltpu.make_async_copy(y_hbm.at[isrc], rows, sem).wait()
            pltpu.sync_copy(rows, acc_sh.at[idst], add=True)

        start(0, isrc_a, idst_a, rows_a, sem_a)

        def body(i, carry):
            start(2 * i + 1, isrc_b, idst_b, rows_b, sem_b)
            finish(isrc_a, idst_a, rows_a, sem_a)
            start(2 * i + 2, isrc_a, idst_a, rows_a, sem_a)
            finish(isrc_b, idst_b, rows_b, sem_b)
            return carry

        lax.fori_loop(0, (nchunks - 1) // 2, body, 0)
        finish(isrc_a, idst_a, rows_a, sem_a)
        plsc.subcore_barrier()
        pltpu.sync_copy(acc_sh.at[pl.ds(s * rpt, rpt)],
                        out_hbm.at[c, pl.ds(s * rpt, rpt)])

    return scatter_kernel


def _make_edge_mlp_kernel(N, E, C):
    epw = E // (NC * NS)
    nchunks = epw // C
    assert nchunks % 2 == 1 and C % 16 == 0
    G = H // 32

    def _make(Npad):
        rpt = Npad // NS

        @functools.partial(
            pl.kernel,
            out_type=jax.ShapeDtypeStruct((E,), jnp.float32),
            mesh=_sc_mesh(),
            scratch_types=[
                pltpu.VMEM_SHARED((Npad, H), jnp.int32),
                pltpu.VMEM((C, H), jnp.int32),
                pltpu.VMEM((C, H), jnp.int32),
                pltpu.VMEM((C, H), jnp.int32),
                pltpu.VMEM((C, H), jnp.int32),
                pltpu.VMEM((C,), jnp.int32),
                pltpu.VMEM((C,), jnp.int32),
                pltpu.VMEM((C,), jnp.int32),
                pltpu.VMEM((C,), jnp.int32),
                pltpu.VMEM((2 * C,), jnp.float32),
                pltpu.VMEM((2 * C,), jnp.float32),
                pltpu.VMEM((C,), jnp.float32),
                pltpu.VMEM((C,), jnp.float32),
                pltpu.VMEM((4, H // 2), jnp.int32),
                pltpu.SemaphoreType.DMA,
                pltpu.SemaphoreType.DMA,
                pltpu.SemaphoreType.DMA,
                pltpu.SemaphoreType.DMA,
                pltpu.SemaphoreType.DMA,
                pltpu.SemaphoreType.DMA,
            ],
            compiler_params=_sc_params(),
        )
        def edge_kernel(pq_hbm, src_hbm, dst_hbm, eaf_hbm, uw_hbm, out_hbm,
                        pq_sh, pa, pb, qa, qb, isrc_a, isrc_b, idst_a, idst_b,
                        ea_a, ea_b, out_a, out_b, uw_v,
                        sp_a, sp_b, sq_a, sq_b, so_a, so_b):
            c = lax.axis_index("c")
            s = lax.axis_index("s")
            wid = c * NS + s
            base = wid * epw
            pltpu.sync_copy(pq_hbm.at[pl.ds(s * rpt, rpt)],
                            pq_sh.at[pl.ds(s * rpt, rpt)])
            pltpu.sync_copy(uw_hbm, uw_v)
            plsc.subcore_barrier()
            lane = lax.iota(jnp.int32, 16)

            def start(k, isrc, idst, p_v, q_v, ea_v, sp, sq):
                pltpu.sync_copy(src_hbm.at[pl.ds(base + k * C, C)], isrc)
                pltpu.sync_copy(dst_hbm.at[pl.ds(base + k * C, C)], idst)
                pltpu.sync_copy(eaf_hbm.at[pl.ds(2 * (base + k * C), 2 * C)],
                                ea_v)
                pltpu.async_copy(pq_sh.at[isrc], p_v, sp)
                pltpu.async_copy(pq_sh.at[idst], q_v, sq)

            def finish(k, kprev, isrc, idst, p_v, q_v, ea_v, out_v, sp, sq, so):
                pltpu.make_async_copy(pq_sh.at[isrc], p_v, sp).wait()
                pltpu.make_async_copy(pq_sh.at[idst], q_v, sq).wait()
                pltpu.make_async_copy(
                    out_v, out_hbm.at[pl.ds(base + kprev * C, C)], so).wait()

                def grp(j2, carry):
                    off = j2 * 16
                    pv0 = ea_v[pl.ds(2 * off, 16)]
                    pv1 = ea_v[pl.ds(2 * off + 16, 16)]
                    acc0 = plsc.bitcast(uw_v[3, pl.ds(0, 16)], jnp.bfloat16)
                    res = jnp.zeros((16,), jnp.float32)
                    for e in range(16):
                        j = j2 * 16 + e
                        pv = pv0 if e < 8 else pv1
                        s0 = pv[(2 * e) % 16]
                        s1 = pv[(2 * e + 1) % 16]
                        s0f = jnp.full((16,), s0, jnp.float32)
                        s1f = jnp.full((16,), s1, jnp.float32)
                        s0b = plsc.pack(s0f, s0f,
                                        format=plsc.PackFormat.INTERLEAVED)
                        s1b = plsc.pack(s1f, s1f,
                                        format=plsc.PackFormat.INTERLEAVED)
                        acc = acc0
                        for g in range(G):
                            slp = pl.ds(g * 16, 16)
                            slq = pl.ds(H // 2 + g * 16, 16)
                            hv = (plsc.bitcast(p_v[j, slp], jnp.bfloat16)
                                  + plsc.bitcast(q_v[j, slq], jnp.bfloat16)
                                  + s0b * plsc.bitcast(uw_v[0, slp],
                                                       jnp.bfloat16)
                                  + s1b * plsc.bitcast(uw_v[1, slp],
                                                       jnp.bfloat16))
                            hv = jnp.maximum(hv, 0)
                            acc = acc + hv * plsc.bitcast(uw_v[2, slp],
                                                          jnp.bfloat16)
                        a_lo, a_hi = plsc.unpack(
                            acc, format=plsc.PackFormat.INTERLEAVED)
                        tot = jnp.sum(a_lo + a_hi)
                        res = jnp.where(lane == e, tot, res)
                    out_v[pl.ds(off, 16)] = res
                    return carry

                lax.fori_loop(0, C // 16, grp, 0)
                pltpu.async_copy(out_v, out_hbm.at[pl.ds(base + k * C, C)], so)

            pltpu.async_copy(out_a, out_hbm.at[pl.ds(base, C)], so_a)
            pltpu.async_copy(out_b, out_hbm.at[pl.ds(base + C, C)], so_b)
            start(0, isrc_a, idst_a, pa, qa, ea_a, sp_a, sq_a)

            def body(i, carry):
                start(2 * i + 1, isrc_b, idst_b, pb, qb, ea_b, sp_b, sq_b)
                finish(2 * i, jnp.maximum(2 * i - 2, 0),
                       isrc_a, idst_a, pa, qa, ea_a, out_a, sp_a, sq_a, so_a)
                start(2 * i + 2, isrc_a, idst_a, pa, qa, ea_a, sp_a, sq_a)
                finish(2 * i + 1, jnp.maximum(2 * i - 1, 1),
                       isrc_b, idst_b, pb, qb, ea_b, out_b, sp_b, sq_b, so_b)
                return carry

            lax.fori_loop(0, (nchunks - 1) // 2, body, 0)
            finish(nchunks - 1, nchunks - 3,
                   isrc_a, idst_a, pa, qa, ea_a, out_a, sp_a, sq_a, so_a)
            pltpu.make_async_copy(
                out_a, out_hbm.at[pl.ds(base + (nchunks - 1) * C, C)],
                so_a).wait()
            pltpu.make_async_copy(
                out_b, out_hbm.at[pl.ds(base + (nchunks - 2) * C, C)],
                so_b).wait()

        return edge_kernel

    return _make


def _make_edge_gather_kernel(N, E, C):
    epw = E // (NC * NS)
    nchunks = epw // C

    assert nchunks % 2 == 1

    @functools.partial(
        pl.kernel,
        out_type=jax.ShapeDtypeStruct((E, H), jnp.float32),
        mesh=_sc_mesh(),
        scratch_types=[
            pltpu.VMEM((C, H), jnp.float32),
            pltpu.VMEM((C, H), jnp.float32),
            pltpu.VMEM((C, H), jnp.float32),
            pltpu.VMEM((C, H), jnp.float32),
            pltpu.VMEM((C,), jnp.int32),
            pltpu.VMEM((C,), jnp.int32),
            pltpu.VMEM((C,), jnp.int32),
            pltpu.VMEM((C,), jnp.int32),
            pltpu.SemaphoreType.DMA,
            pltpu.SemaphoreType.DMA,
            pltpu.SemaphoreType.DMA,
            pltpu.SemaphoreType.DMA,
        ],
    )
    def edge_kernel(p_hbm, q_hbm, src_hbm, dst_hbm, out_hbm,
                    pa, pb, qa, qb, isrc_a, isrc_b, idst_a, idst_b,
                    sp_a, sp_b, sq_a, sq_b):
        c = lax.axis_index("c")
        s = lax.axis_index("s")
        wid = c * NS + s
        base = wid * epw

        def start(k, isrc, idst, p_v, q_v, sp, sq):
            pltpu.sync_copy(src_hbm.at[pl.ds(base + k * C, C)], isrc)
            pltpu.sync_copy(dst_hbm.at[pl.ds(base + k * C, C)], idst)
            pltpu.async_copy(p_hbm.at[isrc], p_v, sp)
            pltpu.async_copy(q_hbm.at[idst], q_v, sq)

        def finish(k, isrc, idst, p_v, q_v, sp, sq):
            pltpu.make_async_copy(p_hbm.at[isrc], p_v, sp).wait()
            pltpu.make_async_copy(q_hbm.at[idst], q_v, sq).wait()

            def add_body(j, acc):
                for g in range(H // 16):
                    sl = pl.ds(g * 16, 16)
                    p_v[j, sl] = p_v[j, sl] + q_v[j, sl]
                return acc

            lax.fori_loop(0, C, add_body, 0)
            pltpu.sync_copy(p_v, out_hbm.at[pl.ds(base + k * C, C)])

        start(0, isrc_a, idst_a, pa, qa, sp_a, sq_a)

        def body(i, carry):
            start(2 * i + 1, isrc_b, idst_b, pb, qb, sp_b, sq_b)
            finish(2 * i, isrc_a, idst_a, pa, qa, sp_a, sq_a)
            start(2 * i + 2, isrc_a, idst_a, pa, qa, sp_a, sq_a)
            finish(2 * i + 1, isrc_b, idst_b, pb, qb, sp_b, sq_b)
            return carry

        lax.fori_loop(0, (nchunks - 1) // 2, body, 0)
        finish(nchunks - 1, isrc_a, idst_a, pa, qa, sp_a, sq_a)

    return edge_kernel


def _tc_pre(x, W_gcn, h_prev, W_hhT, b_hh, deg2):
    N = x.shape[0]
    R = 1000
    grid = (N // R,)

    def body(x_ref, wg_ref, hp_ref, whh_ref, bhh_ref, deg_ref, y_ref, gh_ref):
        deg = deg_ref[0, :, 0] + deg_ref[1, :, 0] + 1.0
        dinv = lax.rsqrt(deg)
        xw = jnp.dot(x_ref[...], wg_ref[...], preferred_element_type=jnp.float32)
        y_ref[...] = xw * dinv[:, None]
        gh_ref[...] = (jnp.dot(hp_ref[...], whh_ref[...],
                               preferred_element_type=jnp.float32)
                       + bhh_ref[...][None, :])

    return pl.pallas_call(
        body,
        grid=grid,
        in_specs=[
            pl.BlockSpec((R, H), lambda i: (i, 0)),
            pl.BlockSpec((H, H), lambda i: (0, 0)),
            pl.BlockSpec((R, H), lambda i: (i, 0)),
            pl.BlockSpec((H, 3 * H), lambda i: (0, 0)),
            pl.BlockSpec((3 * H,), lambda i: (0,)),
            pl.BlockSpec((NC, R, H), lambda i: (0, i, 0)),
        ],
        out_specs=[
            pl.BlockSpec((R, H), lambda i: (i, 0)),
            pl.BlockSpec((R, 3 * H), lambda i: (i, 0)),
        ],
        out_shape=[
            jax.ShapeDtypeStruct((N, H), jnp.float32),
            jax.ShapeDtypeStruct((N, 3 * H), jnp.float32),
        ],
    )(x, W_gcn, h_prev, W_hhT, b_hh, deg2)


def _tc_mid(S2, y, deg2, b_gcn, W_ihT, b_ih, gh, h_prev, W1aT, W1bT, b1):
    N = y.shape[0]
    R = 2000
    grid = (N // R,)

    def body(s2_ref, y_ref, deg_ref, bg_ref, wih_ref, bih_ref, gh_ref,
             hp_ref, w1a_ref, w1b_ref, b1_ref, hn_ref, p_ref, q_ref):
        deg = deg_ref[0, :, 0] + deg_ref[1, :, 0] + 1.0
        dinv = lax.rsqrt(deg)
        ssum = s2_ref[0] + s2_ref[1] + y_ref[...]
        agg = ssum * dinv[:, None] + bg_ref[...][None, :]
        h_curr = jnp.maximum(agg, 0.0)
        gi = (jnp.dot(h_curr, wih_ref[...], preferred_element_type=jnp.float32)
              + bih_ref[...][None, :])
        gh = gh_ref[...]
        r = jax.nn.sigmoid(gi[:, :H] + gh[:, :H])
        z = jax.nn.sigmoid(gi[:, H:2 * H] + gh[:, H:2 * H])
        ng = jnp.tanh(gi[:, 2 * H:] + r * gh[:, 2 * H:])
        hp = hp_ref[...]
        hn = (1.0 - z) * ng + z * hp
        hn_ref[...] = hn
        p_ref[...] = (jnp.dot(hn, w1a_ref[...], preferred_element_type=jnp.float32)
                      + b1_ref[...][None, :]).astype(jnp.bfloat16)
        q_ref[...] = jnp.dot(hn, w1b_ref[...],
                             preferred_element_type=jnp.float32).astype(jnp.bfloat16)

    return pl.pallas_call(
        body,
        grid=grid,
        in_specs=[
            pl.BlockSpec((NC, R, H), lambda i: (0, i, 0)),
            pl.BlockSpec((R, H), lambda i: (i, 0)),
            pl.BlockSpec((NC, R, H), lambda i: (0, i, 0)),
            pl.BlockSpec((H,), lambda i: (0,)),
            pl.BlockSpec((H, 3 * H), lambda i: (0, 0)),
            pl.BlockSpec((3 * H,), lambda i: (0,)),
            pl.BlockSpec((R, 3 * H), lambda i: (i, 0)),
            pl.BlockSpec((R, H), lambda i: (i, 0)),
            pl.BlockSpec((H, H), lambda i: (0, 0)),
            pl.BlockSpec((H, H), lambda i: (0, 0)),
            pl.BlockSpec((H,), lambda i: (0,)),
        ],
        out_specs=[
            pl.BlockSpec((R, H), lambda i: (i, 0)),
            pl.BlockSpec((R, H), lambda i: (i, 0)),
            pl.BlockSpec((R, H), lambda i: (i, 0)),
        ],
        out_shape=[
            jax.ShapeDtypeStruct((N, H), jnp.float32),
            jax.ShapeDtypeStruct((N, H), jnp.bfloat16),
            jax.ShapeDtypeStruct((N, H), jnp.bfloat16),
        ],
    )(S2, y, deg2, b_gcn, W_ihT, b_ih, gh, h_prev, W1aT, W1bT, b1)


def _tc_edge_mlp(Rpq, ea, u01, w2m, diag_m, b2):
    E = Rpq.shape[0]
    B = 512
    grid = (E // B,)

    def body(r_ref, ea_ref, u_ref, w2_ref, m_ref, b2_ref, out_ref):
        eau = jnp.dot(ea_ref[...], u_ref[...], preferred_element_type=jnp.float32)
        hid = jnp.maximum(r_ref[...] + eau, 0.0)
        t = jnp.dot(hid, w2_ref[...], preferred_element_type=jnp.float32)
        v = jnp.sum(t.reshape(B // 64, 64, H) * m_ref[...][None], axis=1)
        out_ref[...] = v[:, :64] + b2_ref[...][0]

    out2d = pl.pallas_call(
        body,
        grid=grid,
        in_specs=[
            pl.BlockSpec((B, H), lambda i: (i, 0)),
            pl.BlockSpec((B, 2), lambda i: (i, 0)),
            pl.BlockSpec((2, H), lambda i: (0, 0)),
            pl.BlockSpec((H, H), lambda i: (0, 0)),
            pl.BlockSpec((64, H), lambda i: (0, 0)),
            pl.BlockSpec((1,), lambda i: (0,)),
        ],
        out_specs=pl.BlockSpec((B // 64, 64), lambda i: (i, 0)),
        out_shape=jax.ShapeDtypeStruct((E // 64, 64), jnp.float32),
    )(Rpq, ea, u01, w2m, diag_m, b2)
    return out2d.reshape(E)


def kernel(x, edge_index, edge_attr, h_prev, W_gcn, b_gcn, W_ih, W_hh,
           b_ih, b_hh, W1, b1, W2, b2):
    N = x.shape[0]
    E = edge_index.shape[1]
    C = 80
    Np = ((N + NS * 8 - 1) // (NS * 8)) * (NS * 8)

    src = edge_index[0]
    dst = edge_index[1]
    onesH = jnp.ones((C, H), jnp.float32)
    zerosH = jnp.zeros((Np // NS, H), jnp.float32)

    deg2 = _make_deg_kernel(Np, E, C)(dst, onesH, zerosH)
    y, gh = _tc_pre(x, W_gcn, h_prev, W_hh.T, b_hh, deg2)
    S2 = _make_scatter_kernel(Np, E, C)(y, src, dst, zerosH)
    h_next, P, Q = _tc_mid(S2, y, deg2, b_gcn, W_ih.T, b_ih, gh, h_prev,
                           W1[:, :H].T, W1[:, H:2 * H].T, b1)
    PQ = jnp.pad(jnp.concatenate([P, Q], axis=1), ((0, Np - N), (0, 0)))
    PQ32 = lax.bitcast_convert_type(PQ.reshape(Np, H, 2), jnp.int32)
    uw = jnp.stack([W1[:, 2 * H], W1[:, 2 * H + 1], W2[0],
                    jnp.full((H,), b2[0] / 32.0, jnp.float32)])
    uw32 = lax.bitcast_convert_type(
        uw.astype(jnp.bfloat16).reshape(4, H // 2, 2), jnp.int32)
    eaf = edge_attr.reshape(2 * E)
    out = _make_edge_mlp_kernel(N, E, C)(Np)(PQ32, src, dst, eaf, uw32)
    return (out, h_next)


# speedup vs baseline: 1.3512x; 1.3512x over previous
import dataclasses
import functools

import jax
import jax.numpy as jnp
from jax import lax
from jax.experimental import pallas as pl
from jax.experimental.pallas import tpu as pltpu
from jax.experimental.pallas import tpu_sc as plsc

NC = 2
NS = 16
H = 128


def _sc_mesh():
    return plsc.VectorSubcoreMesh(
        core_axis_name="c", subcore_axis_name="s", num_cores=NC, num_subcores=NS)


def _sc_params():
    cp = pltpu.CompilerParams()
    if "needs_layout_passes" in pltpu.CompilerParams.__dataclass_fields__:
        cp = dataclasses.replace(cp, needs_layout_passes=False)
    return cp


def _make_deg_kernel(N, E, C):
    epw = E // (NC * NS)
    nchunks = epw // C
    rpt = N // NS

    assert nchunks % 2 == 1

    @functools.partial(
        pl.kernel,
        out_type=jax.ShapeDtypeStruct((NC, N, H), jnp.float32),
        mesh=_sc_mesh(),
        scratch_types=[
            pltpu.VMEM_SHARED((N, H), jnp.float32),
            pltpu.VMEM((C, H), jnp.float32),
            pltpu.VMEM((C,), jnp.int32),
            pltpu.VMEM((C,), jnp.int32),
            pltpu.SemaphoreType.DMA,
            pltpu.SemaphoreType.DMA,
        ],
    )
    def deg_kernel(dst_hbm, ones_hbm, zeros_hbm, out_hbm, acc_sh, ones_v,
                   idx_a, idx_b, sem_a, sem_b):
        c = lax.axis_index("c")
        s = lax.axis_index("s")
        wid = c * NS + s
        pltpu.sync_copy(zeros_hbm, acc_sh.at[pl.ds(s * rpt, rpt)])
        pltpu.sync_copy(ones_hbm, ones_v)
        plsc.subcore_barrier()
        base = wid * epw

        def start(k, idx_v, sem):
            pltpu.sync_copy(dst_hbm.at[pl.ds(base + k * C, C)], idx_v)
            pltpu.async_copy(ones_v, acc_sh.at[idx_v], sem, add=True)

        def finish(idx_v, sem):
            pltpu.make_async_copy(ones_v, acc_sh.at[idx_v], sem).wait()

        start(0, idx_a, sem_a)

        def body(i, carry):
            start(2 * i + 1, idx_b, sem_b)
            finish(idx_a, sem_a)
            start(2 * i + 2, idx_a, sem_a)
            finish(idx_b, sem_b)
            return carry

        lax.fori_loop(0, (nchunks - 1) // 2, body, 0)
        finish(idx_a, sem_a)
        plsc.subcore_barrier()
        pltpu.sync_copy(acc_sh.at[pl.ds(s * rpt, rpt)],
                        out_hbm.at[c, pl.ds(s * rpt, rpt)])

    return deg_kernel


def _make_scatter_kernel(N, E, C):
    epw = E // (NC * NS)
    nchunks = epw // C
    rpt = N // NS

    assert nchunks % 2 == 1

    @functools.partial(
        pl.kernel,
        out_type=jax.ShapeDtypeStruct((NC, N, H), jnp.float32),
        mesh=_sc_mesh(),
        scratch_types=[
            pltpu.VMEM_SHARED((N, H), jnp.float32),
            pltpu.VMEM((C, H), jnp.float32),
            pltpu.VMEM((C, H), jnp.float32),
            pltpu.VMEM((C,), jnp.int32),
            pltpu.VMEM((C,), jnp.int32),
            pltpu.VMEM((C,), jnp.int32),
            pltpu.VMEM((C,), jnp.int32),
            pltpu.SemaphoreType.DMA,
            pltpu.SemaphoreType.DMA,
        ],
    )
    def scatter_kernel(y_hbm, src_hbm, dst_hbm, zeros_hbm, out_hbm,
                       acc_sh, rows_a, rows_b, isrc_a, isrc_b,
                       idst_a, idst_b, sem_a, sem_b):
        c = lax.axis_index("c")
        s = lax.axis_index("s")
        wid = c * NS + s
        pltpu.sync_copy(zeros_hbm, acc_sh.at[pl.ds(s * rpt, rpt)])
        plsc.subcore_barrier()
        base = wid * epw

        def start(k, isrc, idst, rows, sem):
            pltpu.sync_copy(src_hbm.at[pl.ds(base + k * C, C)], isrc)
            pltpu.sync_copy(dst_hbm.at[pl.ds(base + k * C, C)], idst)
            pltpu.async_copy(y_hbm.at[isrc], rows, sem)

        def finish(isrc, idst, rows, sem):
            pltpu.make_async_copy(y_hbm.at[isrc], rows, sem).wait()
            pltpu.sync_copy(rows, acc_sh.at[idst], add=True)

        start(0, isrc_a, idst_a, rows_a, sem_a)

        def body(i, carry):
            start(2 * i + 1, isrc_b, idst_b, rows_b, sem_b)
            finish(isrc_a, idst_a, rows_a, sem_a)
            start(2 * i + 2, isrc_a, idst_a, rows_a, sem_a)
            finish(isrc_b, idst_b, rows_b, sem_b)
            return carry

        lax.fori_loop(0, (nchunks - 1) // 2, body, 0)
        finish(isrc_a, idst_a, rows_a, sem_a)
        plsc.subcore_barrier()
        pltpu.sync_copy(acc_sh.at[pl.ds(s * rpt, rpt)],
                        out_hbm.at[c, pl.ds(s * rpt, rpt)])

    return scatter_kernel


def _make_edge_mlp_kernel(N, E, C):
    epw = E // (NC * NS)
    nchunks = epw // C
    assert nchunks % 2 == 1 and C % 16 == 0
    G = H // 16

    @functools.partial(
        pl.kernel,
        out_type=jax.ShapeDtypeStruct((E,), jnp.float32),
        mesh=_sc_mesh(),
        scratch_types=[
            pltpu.VMEM((C, H), jnp.float32),
            pltpu.VMEM((C, H), jnp.float32),
            pltpu.VMEM((C, H), jnp.float32),
            pltpu.VMEM((C, H), jnp.float32),
            pltpu.VMEM((C,), jnp.int32),
            pltpu.VMEM((C,), jnp.int32),
            pltpu.VMEM((C,), jnp.int32),
            pltpu.VMEM((C,), jnp.int32),
            pltpu.VMEM((2 * epw,), jnp.float32),
            pltpu.VMEM((epw,), jnp.float32),
            pltpu.VMEM((4, H), jnp.float32),
            pltpu.SemaphoreType.DMA,
            pltpu.SemaphoreType.DMA,
            pltpu.SemaphoreType.DMA,
            pltpu.SemaphoreType.DMA,
        ],
        compiler_params=_sc_params(),
    )
    def edge_kernel(p_hbm, q_hbm, src_hbm, dst_hbm, eaf_hbm, uw_hbm, out_hbm,
                    pa, pb, qa, qb, isrc_a, isrc_b, idst_a, idst_b,
                    ea_v, out_v, uw_v,
                    sp_a, sp_b, sq_a, sq_b):
        c = lax.axis_index("c")
        s = lax.axis_index("s")
        wid = c * NS + s
        base = wid * epw
        pltpu.sync_copy(eaf_hbm.at[pl.ds(2 * base, 2 * epw)], ea_v)
        pltpu.sync_copy(uw_hbm, uw_v)
        lane = lax.iota(jnp.int32, 16)

        def start(k, isrc, idst, p_v, q_v, sp, sq):
            pltpu.sync_copy(src_hbm.at[pl.ds(base + k * C, C)], isrc)
            pltpu.sync_copy(dst_hbm.at[pl.ds(base + k * C, C)], idst)
            pltpu.async_copy(p_hbm.at[isrc], p_v, sp)
            pltpu.async_copy(q_hbm.at[idst], q_v, sq)

        def finish(k, isrc, idst, p_v, q_v, sp, sq):
            pltpu.make_async_copy(p_hbm.at[isrc], p_v, sp).wait()
            pltpu.make_async_copy(q_hbm.at[idst], q_v, sq).wait()

            def grp(j2, carry):
                off = k * C + j2 * 16
                pv0 = ea_v[pl.ds(2 * off, 16)]
                pv1 = ea_v[pl.ds(2 * off + 16, 16)]
                acc0 = uw_v[3, pl.ds(0, 16)]
                res = jnp.zeros((16,), jnp.float32)
                for e in range(16):
                    j = j2 * 16 + e
                    pv = pv0 if e < 8 else pv1
                    s0 = pv[(2 * e) % 16]
                    s1 = pv[(2 * e + 1) % 16]
                    acc = acc0
                    for g in range(G):
                        sl = pl.ds(g * 16, 16)
                        hid = (p_v[j, sl] + q_v[j, sl]
                               + s0 * uw_v[0, sl] + s1 * uw_v[1, sl])
                        hid = jnp.maximum(hid, 0.0)
                        acc = acc + hid * uw_v[2, sl]
                    tot = jnp.sum(acc)
                    res = jnp.where(lane == e, tot, res)
                out_v[pl.ds(off, 16)] = res
                return carry

            lax.fori_loop(0, C // 16, grp, 0)

        start(0, isrc_a, idst_a, pa, qa, sp_a, sq_a)

        def body(i, carry):
            start(2 * i + 1, isrc_b, idst_b, pb, qb, sp_b, sq_b)
            finish(2 * i, isrc_a, idst_a, pa, qa, sp_a, sq_a)
            start(2 * i + 2, isrc_a, idst_a, pa, qa, sp_a, sq_a)
            finish(2 * i + 1, isrc_b, idst_b, pb, qb, sp_b, sq_b)
            return carry

        lax.fori_loop(0, (nchunks - 1) // 2, body, 0)
        finish(nchunks - 1, isrc_a, idst_a, pa, qa, sp_a, sq_a)
        pltpu.sync_copy(out_v, out_hbm.at[pl.ds(base, epw)])

    return edge_kernel


def _make_edge_gather_kernel(N, E, C):
    epw = E // (NC * NS)
    nchunks = epw // C

    assert nchunks % 2 == 1

    @functools.partial(
        pl.kernel,
        out_type=jax.ShapeDtypeStruct((E, H), jnp.float32),
        mesh=_sc_mesh(),
        scratch_types=[
            pltpu.VMEM((C, H), jnp.float32),
            pltpu.VMEM((C, H), jnp.float32),
            pltpu.VMEM((C, H), jnp.float32),
            pltpu.VMEM((C, H), jnp.float32),
            pltpu.VMEM((C,), jnp.int32),
            pltpu.VMEM((C,), jnp.int32),
            pltpu.VMEM((C,), jnp.int32),
            pltpu.VMEM((C,), jnp.int32),
            pltpu.SemaphoreType.DMA,
            pltpu.SemaphoreType.DMA,
            pltpu.SemaphoreType.DMA,
            pltpu.SemaphoreType.DMA,
        ],
    )
    def edge_kernel(p_hbm, q_hbm, src_hbm, dst_hbm, out_hbm,
                    pa, pb, qa, qb, isrc_a, isrc_b, idst_a, idst_b,
                    sp_a, sp_b, sq_a, sq_b):
        c = lax.axis_index("c")
        s = lax.axis_index("s")
        wid = c * NS + s
        base = wid * epw

        def start(k, isrc, idst, p_v, q_v, sp, sq):
            pltpu.sync_copy(src_hbm.at[pl.ds(base + k * C, C)], isrc)
            pltpu.sync_copy(dst_hbm.at[pl.ds(base + k * C, C)], idst)
            pltpu.async_copy(p_hbm.at[isrc], p_v, sp)
            pltpu.async_copy(q_hbm.at[idst], q_v, sq)

        def finish(k, isrc, idst, p_v, q_v, sp, sq):
            pltpu.make_async_copy(p_hbm.at[isrc], p_v, sp).wait()
            pltpu.make_async_copy(q_hbm.at[idst], q_v, sq).wait()

            def add_body(j, acc):
                for g in range(H // 16):
                    sl = pl.ds(g * 16, 16)
                    p_v[j, sl] = p_v[j, sl] + q_v[j, sl]
                return acc

            lax.fori_loop(0, C, add_body, 0)
            pltpu.sync_copy(p_v, out_hbm.at[pl.ds(base + k * C, C)])

        start(0, isrc_a, idst_a, pa, qa, sp_a, sq_a)

        def body(i, carry):
            start(2 * i + 1, isrc_b, idst_b, pb, qb, sp_b, sq_b)
            finish(2 * i, isrc_a, idst_a, pa, qa, sp_a, sq_a)
            start(2 * i + 2, isrc_a, idst_a, pa, qa, sp_a, sq_a)
            finish(2 * i + 1, isrc_b, idst_b, pb, qb, sp_b, sq_b)
            return carry

        lax.fori_loop(0, (nchunks - 1) // 2, body, 0)
        finish(nchunks - 1, isrc_a, idst_a, pa, qa, sp_a, sq_a)

    return edge_kernel


def _tc_pre(x, W_gcn, h_prev, W_hhT, b_hh, deg2):
    N = x.shape[0]
    R = 1000
    grid = (N // R,)

    def body(x_ref, wg_ref, hp_ref, whh_ref, bhh_ref, deg_ref, y_ref, gh_ref):
        deg = deg_ref[0, :, 0] + deg_ref[1, :, 0] + 1.0
        dinv = lax.rsqrt(deg)
        xw = jnp.dot(x_ref[...], wg_ref[...], preferred_element_type=jnp.float32)
        y_ref[...] = xw * dinv[:, None]
        gh_ref[...] = (jnp.dot(hp_ref[...], whh_ref[...],
                               preferred_element_type=jnp.float32)
                       + bhh_ref[...][None, :])

    return pl.pallas_call(
        body,
        grid=grid,
        in_specs=[
            pl.BlockSpec((R, H), lambda i: (i, 0)),
            pl.BlockSpec((H, H), lambda i: (0, 0)),
            pl.BlockSpec((R, H), lambda i: (i, 0)),
            pl.BlockSpec((H, 3 * H), lambda i: (0, 0)),
            pl.BlockSpec((3 * H,), lambda i: (0,)),
            pl.BlockSpec((NC, R, H), lambda i: (0, i, 0)),
        ],
        out_specs=[
            pl.BlockSpec((R, H), lambda i: (i, 0)),
            pl.BlockSpec((R, 3 * H), lambda i: (i, 0)),
        ],
        out_shape=[
            jax.ShapeDtypeStruct((N, H), jnp.float32),
            jax.ShapeDtypeStruct((N, 3 * H), jnp.float32),
        ],
    )(x, W_gcn, h_prev, W_hhT, b_hh, deg2)


def _tc_mid(S2, y, deg2, b_gcn, W_ihT, b_ih, gh, h_prev, W1aT, W1bT, b1):
    N = y.shape[0]
    R = 2000
    grid = (N // R,)

    def body(s2_ref, y_ref, deg_ref, bg_ref, wih_ref, bih_ref, gh_ref,
             hp_ref, w1a_ref, w1b_ref, b1_ref, hn_ref, p_ref, q_ref):
        deg = deg_ref[0, :, 0] + deg_ref[1, :, 0] + 1.0
        dinv = lax.rsqrt(deg)
        ssum = s2_ref[0] + s2_ref[1] + y_ref[...]
        agg = ssum * dinv[:, None] + bg_ref[...][None, :]
        h_curr = jnp.maximum(agg, 0.0)
        gi = (jnp.dot(h_curr, wih_ref[...], preferred_element_type=jnp.float32)
              + bih_ref[...][None, :])
        gh = gh_ref[...]
        r = jax.nn.sigmoid(gi[:, :H] + gh[:, :H])
        z = jax.nn.sigmoid(gi[:, H:2 * H] + gh[:, H:2 * H])
        ng = jnp.tanh(gi[:, 2 * H:] + r * gh[:, 2 * H:])
        hp = hp_ref[...]
        hn = (1.0 - z) * ng + z * hp
        hn_ref[...] = hn
        p_ref[...] = (jnp.dot(hn, w1a_ref[...], preferred_element_type=jnp.float32)
                      + b1_ref[...][None, :])
        q_ref[...] = jnp.dot(hn, w1b_ref[...], preferred_element_type=jnp.float32)

    return pl.pallas_call(
        body,
        grid=grid,
        in_specs=[
            pl.BlockSpec((NC, R, H), lambda i: (0, i, 0)),
            pl.BlockSpec((R, H), lambda i: (i, 0)),
            pl.BlockSpec((NC, R, H), lambda i: (0, i, 0)),
            pl.BlockSpec((H,), lambda i: (0,)),
            pl.BlockSpec((H, 3 * H), lambda i: (0, 0)),
            pl.BlockSpec((3 * H,), lambda i: (0,)),
            pl.BlockSpec((R, 3 * H), lambda i: (i, 0)),
            pl.BlockSpec((R, H), lambda i: (i, 0)),
            pl.BlockSpec((H, H), lambda i: (0, 0)),
            pl.BlockSpec((H, H), lambda i: (0, 0)),
            pl.BlockSpec((H,), lambda i: (0,)),
        ],
        out_specs=[
            pl.BlockSpec((R, H), lambda i: (i, 0)),
            pl.BlockSpec((R, H), lambda i: (i, 0)),
            pl.BlockSpec((R, H), lambda i: (i, 0)),
        ],
        out_shape=[
            jax.ShapeDtypeStruct((N, H), jnp.float32),
            jax.ShapeDtypeStruct((N, H), jnp.float32),
            jax.ShapeDtypeStruct((N, H), jnp.float32),
        ],
    )(S2, y, deg2, b_gcn, W_ihT, b_ih, gh, h_prev, W1aT, W1bT, b1)


def _tc_edge_mlp(Rpq, ea, u01, w2m, diag_m, b2):
    E = Rpq.shape[0]
    B = 512
    grid = (E // B,)

    def body(r_ref, ea_ref, u_ref, w2_ref, m_ref, b2_ref, out_ref):
        eau = jnp.dot(ea_ref[...], u_ref[...], preferred_element_type=jnp.float32)
        hid = jnp.maximum(r_ref[...] + eau, 0.0)
        t = jnp.dot(hid, w2_ref[...], preferred_element_type=jnp.float32)
        v = jnp.sum(t.reshape(B // 64, 64, H) * m_ref[...][None], axis=1)
        out_ref[...] = v[:, :64] + b2_ref[...][0]

    out2d = pl.pallas_call(
        body,
        grid=grid,
        in_specs=[
            pl.BlockSpec((B, H), lambda i: (i, 0)),
            pl.BlockSpec((B, 2), lambda i: (i, 0)),
            pl.BlockSpec((2, H), lambda i: (0, 0)),
            pl.BlockSpec((H, H), lambda i: (0, 0)),
            pl.BlockSpec((64, H), lambda i: (0, 0)),
            pl.BlockSpec((1,), lambda i: (0,)),
        ],
        out_specs=pl.BlockSpec((B // 64, 64), lambda i: (i, 0)),
        out_shape=jax.ShapeDtypeStruct((E // 64, 64), jnp.float32),
    )(Rpq, ea, u01, w2m, diag_m, b2)
    return out2d.reshape(E)


def kernel(x, edge_index, edge_attr, h_prev, W_gcn, b_gcn, W_ih, W_hh,
           b_ih, b_hh, W1, b1, W2, b2):
    N = x.shape[0]
    E = edge_index.shape[1]
    C = 80
    Np = ((N + NS * 8 - 1) // (NS * 8)) * (NS * 8)

    src = edge_index[0]
    dst = edge_index[1]
    onesH = jnp.ones((C, H), jnp.float32)
    zerosH = jnp.zeros((Np // NS, H), jnp.float32)

    deg2 = _make_deg_kernel(Np, E, C)(dst, onesH, zerosH)
    y, gh = _tc_pre(x, W_gcn, h_prev, W_hh.T, b_hh, deg2)
    S2 = _make_scatter_kernel(Np, E, C)(y, src, dst, zerosH)
    h_next, P, Q = _tc_mid(S2, y, deg2, b_gcn, W_ih.T, b_ih, gh, h_prev,
                           W1[:, :H].T, W1[:, H:2 * H].T, b1)
    uw = jnp.stack([W1[:, 2 * H], W1[:, 2 * H + 1], W2[0],
                    jnp.full((H,), b2[0] / 16.0, jnp.float32)])
    eaf = edge_attr.reshape(2 * E)
    out = _make_edge_mlp_kernel(N, E, C)(P, Q, src, dst, eaf, uw)
    return (out, h_next)


# speedup vs baseline: 1.4596x; 1.0802x over previous
import dataclasses
import functools

import jax
import jax.numpy as jnp
from jax import lax
from jax.experimental import pallas as pl
from jax.experimental.pallas import tpu as pltpu
from jax.experimental.pallas import tpu_sc as plsc

NC = 2
NS = 16
H = 128


def _sc_mesh():
    return plsc.VectorSubcoreMesh(
        core_axis_name="c", subcore_axis_name="s", num_cores=NC, num_subcores=NS)


def _sc_params():
    cp = pltpu.CompilerParams()
    if "needs_layout_passes" in pltpu.CompilerParams.__dataclass_fields__:
        cp = dataclasses.replace(cp, needs_layout_passes=False)
    return cp


def _make_deg_kernel(N, E, C):
    epw = E // (NC * NS)
    nchunks = epw // C
    rpt = N // NS

    assert nchunks % 2 == 1

    @functools.partial(
        pl.kernel,
        out_type=jax.ShapeDtypeStruct((NC, N, H), jnp.float32),
        mesh=_sc_mesh(),
        scratch_types=[
            pltpu.VMEM_SHARED((N, H), jnp.float32),
            pltpu.VMEM((C, H), jnp.float32),
            pltpu.VMEM((C,), jnp.int32),
            pltpu.VMEM((C,), jnp.int32),
            pltpu.SemaphoreType.DMA,
            pltpu.SemaphoreType.DMA,
        ],
    )
    def deg_kernel(dst_hbm, ones_hbm, zeros_hbm, out_hbm, acc_sh, ones_v,
                   idx_a, idx_b, sem_a, sem_b):
        c = lax.axis_index("c")
        s = lax.axis_index("s")
        wid = c * NS + s
        pltpu.sync_copy(zeros_hbm, acc_sh.at[pl.ds(s * rpt, rpt)])
        pltpu.sync_copy(ones_hbm, ones_v)
        plsc.subcore_barrier()
        base = wid * epw

        def start(k, idx_v, sem):
            pltpu.sync_copy(dst_hbm.at[pl.ds(base + k * C, C)], idx_v)
            pltpu.async_copy(ones_v, acc_sh.at[idx_v], sem, add=True)

        def finish(idx_v, sem):
            pltpu.make_async_copy(ones_v, acc_sh.at[idx_v], sem).wait()

        start(0, idx_a, sem_a)

        def body(i, carry):
            start(2 * i + 1, idx_b, sem_b)
            finish(idx_a, sem_a)
            start(2 * i + 2, idx_a, sem_a)
            finish(idx_b, sem_b)
            return carry

        lax.fori_loop(0, (nchunks - 1) // 2, body, 0)
        finish(idx_a, sem_a)
        plsc.subcore_barrier()
        pltpu.sync_copy(acc_sh.at[pl.ds(s * rpt, rpt)],
                        out_hbm.at[c, pl.ds(s * rpt, rpt)])

    return deg_kernel


def _make_scatter_kernel(N, E, C):
    epw = E // (NC * NS)
    nchunks = epw // C
    rpt = N // NS

    assert nchunks % 2 == 1

    @functools.partial(
        pl.kernel,
        out_type=jax.ShapeDtypeStruct((NC, N, H), jnp.float32),
        mesh=_sc_mesh(),
        scratch_types=[
            pltpu.VMEM_SHARED((N, H), jnp.float32),
            pltpu.VMEM((C, H), jnp.float32),
            pltpu.VMEM((C, H), jnp.float32),
            pltpu.VMEM((C,), jnp.int32),
            pltpu.VMEM((C,), jnp.int32),
            pltpu.VMEM((C,), jnp.int32),
            pltpu.VMEM((C,), jnp.int32),
            pltpu.SemaphoreType.DMA,
            pltpu.SemaphoreType.DMA,
        ],
    )
    def scatter_kernel(y_hbm, src_hbm, dst_hbm, zeros_hbm, out_hbm,
                       acc_sh, rows_a, rows_b, isrc_a, isrc_b,
                       idst_a, idst_b, sem_a, sem_b):
        c = lax.axis_index("c")
        s = lax.axis_index("s")
        wid = c * NS + s
        pltpu.sync_copy(zeros_hbm, acc_sh.at[pl.ds(s * rpt, rpt)])
        plsc.subcore_barrier()
        base = wid * epw

        def start(k, isrc, idst, rows, sem):
            pltpu.sync_copy(src_hbm.at[pl.ds(base + k * C, C)], isrc)
            pltpu.sync_copy(dst_hbm.at[pl.ds(base + k * C, C)], idst)
            pltpu.async_copy(y_hbm.at[isrc], rows, sem)

        def finish(isrc, idst, rows, sem):
            pltpu.make_async_copy(y_hbm.at[isrc], rows, sem).wait()
            pltpu.sync_copy(rows, acc_sh.at[idst], add=True)

        start(0, isrc_a, idst_a, rows_a, sem_a)

        def body(i, carry):
            start(2 * i + 1, isrc_b, idst_b, rows_b, sem_b)
            finish(isrc_a, idst_a, rows_a, sem_a)
            start(2 * i + 2, isrc_a, idst_a, rows_a, sem_a)
            finish(isrc_b, idst_b, rows_b, sem_b)
            return carry

        lax.fori_loop(0, (nchunks - 1) // 2, body, 0)
        finish(isrc_a, idst_a, rows_a, sem_a)
        plsc.subcore_barrier()
        pltpu.sync_copy(acc_sh.at[pl.ds(s * rpt, rpt)],
                        out_hbm.at[c, pl.ds(s * rpt, rpt)])

    return scatter_kernel


def _make_edge_mlp_kernel(N, E, C):
    epw = E // (NC * NS)
    nchunks = epw // C
    assert nchunks % 2 == 1 and C % 16 == 0
    G = H // 16

    @functools.partial(
        pl.kernel,
        out_type=jax.ShapeDtypeStruct((E,), jnp.float32),
        mesh=_sc_mesh(),
        scratch_types=[
            pltpu.VMEM((C, H), jnp.float32),
            pltpu.VMEM((C, H), jnp.float32),
            pltpu.VMEM((C, H), jnp.float32),
            pltpu.VMEM((C, H), jnp.float32),
            pltpu.VMEM((C,), jnp.int32),
            pltpu.VMEM((C,), jnp.int32),
            pltpu.VMEM((C,), jnp.int32),
            pltpu.VMEM((C,), jnp.int32),
            pltpu.VMEM((epw,), jnp.float32),
            pltpu.VMEM((epw,), jnp.float32),
            pltpu.VMEM((epw,), jnp.float32),
            pltpu.VMEM((4, H), jnp.float32),
            pltpu.SemaphoreType.DMA,
            pltpu.SemaphoreType.DMA,
            pltpu.SemaphoreType.DMA,
            pltpu.SemaphoreType.DMA,
        ],
        compiler_params=_sc_params(),
    )
    def edge_kernel(p_hbm, q_hbm, src_hbm, dst_hbm, ea0_hbm, ea1_hbm,
                    uw_hbm, out_hbm,
                    pa, pb, qa, qb, isrc_a, isrc_b, idst_a, idst_b,
                    ea0_v, ea1_v, out_v, uw_v,
                    sp_a, sp_b, sq_a, sq_b):
        c = lax.axis_index("c")
        s = lax.axis_index("s")
        wid = c * NS + s
        base = wid * epw
        pltpu.sync_copy(ea0_hbm.at[pl.ds(base, epw)], ea0_v)
        pltpu.sync_copy(ea1_hbm.at[pl.ds(base, epw)], ea1_v)
        pltpu.sync_copy(uw_hbm, uw_v)
        lane = lax.iota(jnp.int32, 16)

        def start(k, isrc, idst, p_v, q_v, sp, sq):
            pltpu.sync_copy(src_hbm.at[pl.ds(base + k * C, C)], isrc)
            pltpu.sync_copy(dst_hbm.at[pl.ds(base + k * C, C)], idst)
            pltpu.async_copy(p_hbm.at[isrc], p_v, sp)
            pltpu.async_copy(q_hbm.at[idst], q_v, sq)

        def finish(k, isrc, idst, p_v, q_v, sp, sq):
            pltpu.make_async_copy(p_hbm.at[isrc], p_v, sp).wait()
            pltpu.make_async_copy(q_hbm.at[idst], q_v, sq).wait()

            def grp(j2, carry):
                off = k * C + j2 * 16
                e0v = ea0_v[pl.ds(off, 16)]
                e1v = ea1_v[pl.ds(off, 16)]
                acc0 = uw_v[3, pl.ds(0, 16)]
                res = jnp.zeros((16,), jnp.float32)
                for e in range(16):
                    j = j2 * 16 + e
                    s0 = e0v[e]
                    s1 = e1v[e]
                    acc = acc0
                    for g in range(G):
                        sl = pl.ds(g * 16, 16)
                        hid = (p_v[j, sl] + q_v[j, sl]
                               + s0 * uw_v[0, sl] + s1 * uw_v[1, sl])
                        hid = jnp.maximum(hid, 0.0)
                        acc = acc + hid * uw_v[2, sl]
                    tot = jnp.sum(acc)
                    res = jnp.where(lane == e, tot, res)
                out_v[pl.ds(off, 16)] = res
                return carry

            lax.fori_loop(0, C // 16, grp, 0)

        start(0, isrc_a, idst_a, pa, qa, sp_a, sq_a)

        def body(i, carry):
            start(2 * i + 1, isrc_b, idst_b, pb, qb, sp_b, sq_b)
            finish(2 * i, isrc_a, idst_a, pa, qa, sp_a, sq_a)
            start(2 * i + 2, isrc_a, idst_a, pa, qa, sp_a, sq_a)
            finish(2 * i + 1, isrc_b, idst_b, pb, qb, sp_b, sq_b)
            return carry

        lax.fori_loop(0, (nchunks - 1) // 2, body, 0)
        finish(nchunks - 1, isrc_a, idst_a, pa, qa, sp_a, sq_a)
        pltpu.sync_copy(out_v, out_hbm.at[pl.ds(base, epw)])

    return edge_kernel


def _make_edge_gather_kernel(N, E, C):
    epw = E // (NC * NS)
    nchunks = epw // C

    assert nchunks % 2 == 1

    @functools.partial(
        pl.kernel,
        out_type=jax.ShapeDtypeStruct((E, H), jnp.float32),
        mesh=_sc_mesh(),
        scratch_types=[
            pltpu.VMEM((C, H), jnp.float32),
            pltpu.VMEM((C, H), jnp.float32),
            pltpu.VMEM((C, H), jnp.float32),
            pltpu.VMEM((C, H), jnp.float32),
            pltpu.VMEM((C,), jnp.int32),
            pltpu.VMEM((C,), jnp.int32),
            pltpu.VMEM((C,), jnp.int32),
            pltpu.VMEM((C,), jnp.int32),
            pltpu.SemaphoreType.DMA,
            pltpu.SemaphoreType.DMA,
            pltpu.SemaphoreType.DMA,
            pltpu.SemaphoreType.DMA,
        ],
    )
    def edge_kernel(p_hbm, q_hbm, src_hbm, dst_hbm, out_hbm,
                    pa, pb, qa, qb, isrc_a, isrc_b, idst_a, idst_b,
                    sp_a, sp_b, sq_a, sq_b):
        c = lax.axis_index("c")
        s = lax.axis_index("s")
        wid = c * NS + s
        base = wid * epw

        def start(k, isrc, idst, p_v, q_v, sp, sq):
            pltpu.sync_copy(src_hbm.at[pl.ds(base + k * C, C)], isrc)
            pltpu.sync_copy(dst_hbm.at[pl.ds(base + k * C, C)], idst)
            pltpu.async_copy(p_hbm.at[isrc], p_v, sp)
            pltpu.async_copy(q_hbm.at[idst], q_v, sq)

        def finish(k, isrc, idst, p_v, q_v, sp, sq):
            pltpu.make_async_copy(p_hbm.at[isrc], p_v, sp).wait()
            pltpu.make_async_copy(q_hbm.at[idst], q_v, sq).wait()

            def add_body(j, acc):
                for g in range(H // 16):
                    sl = pl.ds(g * 16, 16)
                    p_v[j, sl] = p_v[j, sl] + q_v[j, sl]
                return acc

            lax.fori_loop(0, C, add_body, 0)
            pltpu.sync_copy(p_v, out_hbm.at[pl.ds(base + k * C, C)])

        start(0, isrc_a, idst_a, pa, qa, sp_a, sq_a)

        def body(i, carry):
            start(2 * i + 1, isrc_b, idst_b, pb, qb, sp_b, sq_b)
            finish(2 * i, isrc_a, idst_a, pa, qa, sp_a, sq_a)
            start(2 * i + 2, isrc_a, idst_a, pa, qa, sp_a, sq_a)
            finish(2 * i + 1, isrc_b, idst_b, pb, qb, sp_b, sq_b)
            return carry

        lax.fori_loop(0, (nchunks - 1) // 2, body, 0)
        finish(nchunks - 1, isrc_a, idst_a, pa, qa, sp_a, sq_a)

    return edge_kernel


def _tc_pre(x, W_gcn, h_prev, W_hhT, b_hh, deg2):
    N = x.shape[0]
    R = 1000
    grid = (N // R,)

    def body(x_ref, wg_ref, hp_ref, whh_ref, bhh_ref, deg_ref, y_ref, gh_ref):
        deg = deg_ref[0, :, 0] + deg_ref[1, :, 0] + 1.0
        dinv = lax.rsqrt(deg)
        xw = jnp.dot(x_ref[...], wg_ref[...], preferred_element_type=jnp.float32)
        y_ref[...] = xw * dinv[:, None]
        gh_ref[...] = (jnp.dot(hp_ref[...], whh_ref[...],
                               preferred_element_type=jnp.float32)
                       + bhh_ref[...][None, :])

    return pl.pallas_call(
        body,
        grid=grid,
        in_specs=[
            pl.BlockSpec((R, H), lambda i: (i, 0)),
            pl.BlockSpec((H, H), lambda i: (0, 0)),
            pl.BlockSpec((R, H), lambda i: (i, 0)),
            pl.BlockSpec((H, 3 * H), lambda i: (0, 0)),
            pl.BlockSpec((3 * H,), lambda i: (0,)),
            pl.BlockSpec((NC, R, H), lambda i: (0, i, 0)),
        ],
        out_specs=[
            pl.BlockSpec((R, H), lambda i: (i, 0)),
            pl.BlockSpec((R, 3 * H), lambda i: (i, 0)),
        ],
        out_shape=[
            jax.ShapeDtypeStruct((N, H), jnp.float32),
            jax.ShapeDtypeStruct((N, 3 * H), jnp.float32),
        ],
    )(x, W_gcn, h_prev, W_hhT, b_hh, deg2)


def _tc_mid(S2, y, deg2, b_gcn, W_ihT, b_ih, gh, h_prev, W1aT, W1bT, b1):
    N = y.shape[0]
    R = 1000
    grid = (N // R,)

    def body(s2_ref, y_ref, deg_ref, bg_ref, wih_ref, bih_ref, gh_ref,
             hp_ref, w1a_ref, w1b_ref, b1_ref, hn_ref, p_ref, q_ref):
        deg = deg_ref[0, :, 0] + deg_ref[1, :, 0] + 1.0
        dinv = lax.rsqrt(deg)
        ssum = s2_ref[0] + s2_ref[1] + y_ref[...]
        agg = ssum * dinv[:, None] + bg_ref[...][None, :]
        h_curr = jnp.maximum(agg, 0.0)
        gi = (jnp.dot(h_curr, wih_ref[...], preferred_element_type=jnp.float32)
              + bih_ref[...][None, :])
        gh = gh_ref[...]
        r = jax.nn.sigmoid(gi[:, :H] + gh[:, :H])
        z = jax.nn.sigmoid(gi[:, H:2 * H] + gh[:, H:2 * H])
        ng = jnp.tanh(gi[:, 2 * H:] + r * gh[:, 2 * H:])
        hp = hp_ref[...]
        hn = (1.0 - z) * ng + z * hp
        hn_ref[...] = hn
        p_ref[...] = (jnp.dot(hn, w1a_ref[...], preferred_element_type=jnp.float32)
                      + b1_ref[...][None, :])
        q_ref[...] = jnp.dot(hn, w1b_ref[...], preferred_element_type=jnp.float32)

    return pl.pallas_call(
        body,
        grid=grid,
        in_specs=[
            pl.BlockSpec((NC, R, H), lambda i: (0, i, 0)),
            pl.BlockSpec((R, H), lambda i: (i, 0)),
            pl.BlockSpec((NC, R, H), lambda i: (0, i, 0)),
            pl.BlockSpec((H,), lambda i: (0,)),
            pl.BlockSpec((H, 3 * H), lambda i: (0, 0)),
            pl.BlockSpec((3 * H,), lambda i: (0,)),
            pl.BlockSpec((R, 3 * H), lambda i: (i, 0)),
            pl.BlockSpec((R, H), lambda i: (i, 0)),
            pl.BlockSpec((H, H), lambda i: (0, 0)),
            pl.BlockSpec((H, H), lambda i: (0, 0)),
            pl.BlockSpec((H,), lambda i: (0,)),
        ],
        out_specs=[
            pl.BlockSpec((R, H), lambda i: (i, 0)),
            pl.BlockSpec((R, H), lambda i: (i, 0)),
            pl.BlockSpec((R, H), lambda i: (i, 0)),
        ],
        out_shape=[
            jax.ShapeDtypeStruct((N, H), jnp.float32),
            jax.ShapeDtypeStruct((N, H), jnp.float32),
            jax.ShapeDtypeStruct((N, H), jnp.float32),
        ],
    )(S2, y, deg2, b_gcn, W_ihT, b_ih, gh, h_prev, W1aT, W1bT, b1)


def _tc_edge_mlp(Rpq, ea, u01, w2m, diag_m, b2):
    E = Rpq.shape[0]
    B = 512
    grid = (E // B,)

    def body(r_ref, ea_ref, u_ref, w2_ref, m_ref, b2_ref, out_ref):
        eau = jnp.dot(ea_ref[...], u_ref[...], preferred_element_type=jnp.float32)
        hid = jnp.maximum(r_ref[...] + eau, 0.0)
        t = jnp.dot(hid, w2_ref[...], preferred_element_type=jnp.float32)
        v = jnp.sum(t.reshape(B // 64, 64, H) * m_ref[...][None], axis=1)
        out_ref[...] = v[:, :64] + b2_ref[...][0]

    out2d = pl.pallas_call(
        body,
        grid=grid,
        in_specs=[
            pl.BlockSpec((B, H), lambda i: (i, 0)),
            pl.BlockSpec((B, 2), lambda i: (i, 0)),
            pl.BlockSpec((2, H), lambda i: (0, 0)),
            pl.BlockSpec((H, H), lambda i: (0, 0)),
            pl.BlockSpec((64, H), lambda i: (0, 0)),
            pl.BlockSpec((1,), lambda i: (0,)),
        ],
        out_specs=pl.BlockSpec((B // 64, 64), lambda i: (i, 0)),
        out_shape=jax.ShapeDtypeStruct((E // 64, 64), jnp.float32),
    )(Rpq, ea, u01, w2m, diag_m, b2)
    return out2d.reshape(E)


def kernel(x, edge_index, edge_attr, h_prev, W_gcn, b_gcn, W_ih, W_hh,
           b_ih, b_hh, W1, b1, W2, b2):
    N = x.shape[0]
    E = edge_index.shape[1]
    C = 80
    Np = ((N + NS * 8 - 1) // (NS * 8)) * (NS * 8)

    src = edge_index[0]
    dst = edge_index[1]
    onesH = jnp.ones((C, H), jnp.float32)
    zerosH = jnp.zeros((Np // NS, H), jnp.float32)

    deg2 = _make_deg_kernel(Np, E, C)(dst, onesH, zerosH)
    y, gh = _tc_pre(x, W_gcn, h_prev, W_hh.T, b_hh, deg2)
    S2 = _make_scatter_kernel(Np, E, C)(y, src, dst, zerosH)
    h_next, P, Q = _tc_mid(S2, y, deg2, b_gcn, W_ih.T, b_ih, gh, h_prev,
                           W1[:, :H].T, W1[:, H:2 * H].T, b1)
    uw = jnp.stack([W1[:, 2 * H], W1[:, 2 * H + 1], W2[0],
                    jnp.full((H,), b2[0] / 16.0, jnp.float32)])
    out = _make_edge_mlp_kernel(N, E, C)(P, Q, src, dst,
                                         edge_attr[:, 0], edge_attr[:, 1], uw)
    return (out, h_next)


# speedup vs baseline: 1.4599x; 1.0002x over previous
import dataclasses
import functools

import jax
import jax.numpy as jnp
from jax import lax
from jax.experimental import pallas as pl
from jax.experimental.pallas import tpu as pltpu
from jax.experimental.pallas import tpu_sc as plsc

NC = 2
NS = 16
H = 128


def _sc_mesh():
    return plsc.VectorSubcoreMesh(
        core_axis_name="c", subcore_axis_name="s", num_cores=NC, num_subcores=NS)


def _sc_params():
    cp = pltpu.CompilerParams()
    if "needs_layout_passes" in pltpu.CompilerParams.__dataclass_fields__:
        cp = dataclasses.replace(cp, needs_layout_passes=False)
    return cp


def _make_deg_kernel(N, E, C):
    epw = E // (NC * NS)
    nchunks = epw // C
    rpt = N // NS

    assert nchunks % 2 == 1

    @functools.partial(
        pl.kernel,
        out_type=jax.ShapeDtypeStruct((NC, N, H), jnp.float32),
        mesh=_sc_mesh(),
        scratch_types=[
            pltpu.VMEM_SHARED((N, H), jnp.float32),
            pltpu.VMEM((C, H), jnp.float32),
            pltpu.VMEM((C,), jnp.int32),
            pltpu.VMEM((C,), jnp.int32),
            pltpu.SemaphoreType.DMA,
            pltpu.SemaphoreType.DMA,
        ],
    )
    def deg_kernel(dst_hbm, ones_hbm, zeros_hbm, out_hbm, acc_sh, ones_v,
                   idx_a, idx_b, sem_a, sem_b):
        c = lax.axis_index("c")
        s = lax.axis_index("s")
        wid = c * NS + s
        pltpu.sync_copy(zeros_hbm, acc_sh.at[pl.ds(s * rpt, rpt)])
        pltpu.sync_copy(ones_hbm, ones_v)
        plsc.subcore_barrier()
        base = wid * epw

        def start(k, idx_v, sem):
            pltpu.sync_copy(dst_hbm.at[pl.ds(base + k * C, C)], idx_v)
            pltpu.async_copy(ones_v, acc_sh.at[idx_v], sem, add=True)

        def finish(idx_v, sem):
            pltpu.make_async_copy(ones_v, acc_sh.at[idx_v], sem).wait()

        start(0, idx_a, sem_a)

        def body(i, carry):
            start(2 * i + 1, idx_b, sem_b)
            finish(idx_a, sem_a)
            start(2 * i + 2, idx_a, sem_a)
            finish(idx_b, sem_b)
            return carry

        lax.fori_loop(0, (nchunks - 1) // 2, body, 0)
        finish(idx_a, sem_a)
        plsc.subcore_barrier()
        pltpu.sync_copy(acc_sh.at[pl.ds(s * rpt, rpt)],
                        out_hbm.at[c, pl.ds(s * rpt, rpt)])

    return deg_kernel


def _make_scatter_kernel(N, E, C):
    epw = E // (NC * NS)
    nchunks = epw // C
    rpt = N // NS

    assert nchunks % 2 == 1

    @functools.partial(
        pl.kernel,
        out_type=jax.ShapeDtypeStruct((NC, N, H), jnp.float32),
        mesh=_sc_mesh(),
        scratch_types=[
            pltpu.VMEM_SHARED((N, H), jnp.float32),
            pltpu.VMEM((C, H), jnp.float32),
            pltpu.VMEM((C, H), jnp.float32),
            pltpu.VMEM((C,), jnp.int32),
            pltpu.VMEM((C,), jnp.int32),
            pltpu.VMEM((C,), jnp.int32),
            pltpu.VMEM((C,), jnp.int32),
            pltpu.SemaphoreType.DMA,
            pltpu.SemaphoreType.DMA,
        ],
    )
    def scatter_kernel(y_hbm, src_hbm, dst_hbm, zeros_hbm, out_hbm,
                       acc_sh, rows_a, rows_b, isrc_a, isrc_b,
                       idst_a, idst_b, sem_a, sem_b):
        c = lax.axis_index("c")
        s = lax.axis_index("s")
        wid = c * NS + s
        pltpu.sync_copy(zeros_hbm, acc_sh.at[pl.ds(s * rpt, rpt)])
        plsc.subcore_barrier()
        base = wid * epw

        def start(k, isrc, idst, rows, sem):
            pltpu.sync_copy(src_hbm.at[pl.ds(base + k * C, C)], isrc)
            pltpu.sync_copy(dst_hbm.at[pl.ds(base + k * C, C)], idst)
            pltpu.async_copy(y_hbm.at[isrc], rows, sem)

        def finish(isrc, idst, rows, sem):
            pltpu.make_async_copy(y_hbm.at[isrc], rows, sem).wait()
            pltpu.sync_copy(rows, acc_sh.at[idst], add=True)

        start(0, isrc_a, idst_a, rows_a, sem_a)

        def body(i, carry):
            start(2 * i + 1, isrc_b, idst_b, rows_b, sem_b)
            finish(isrc_a, idst_a, rows_a, sem_a)
            start(2 * i + 2, isrc_a, idst_a, rows_a, sem_a)
            finish(isrc_b, idst_b, rows_b, sem_b)
            return carry

        lax.fori_loop(0, (nchunks - 1) // 2, body, 0)
        finish(isrc_a, idst_a, rows_a, sem_a)
        plsc.subcore_barrier()
        pltpu.sync_copy(acc_sh.at[pl.ds(s * rpt, rpt)],
                        out_hbm.at[c, pl.ds(s * rpt, rpt)])

    return scatter_kernel


def _make_edge_mlp_kernel(N, E, C):
    epw = E // (NC * NS)
    nchunks = epw // C
    assert nchunks % 2 == 1 and C % 16 == 0
    G = H // 16

    @functools.partial(
        pl.kernel,
        out_type=jax.ShapeDtypeStruct((E,), jnp.float32),
        mesh=_sc_mesh(),
        scratch_types=[
            pltpu.VMEM((C, H), jnp.float32),
            pltpu.VMEM((C, H), jnp.float32),
            pltpu.VMEM((C, H), jnp.float32),
            pltpu.VMEM((C, H), jnp.float32),
            pltpu.VMEM((C,), jnp.int32),
            pltpu.VMEM((C,), jnp.int32),
            pltpu.VMEM((C,), jnp.int32),
            pltpu.VMEM((C,), jnp.int32),
            pltpu.VMEM((epw,), jnp.float32),
            pltpu.VMEM((epw,), jnp.float32),
            pltpu.VMEM((epw,), jnp.float32),
            pltpu.VMEM((4, H), jnp.float32),
            pltpu.SemaphoreType.DMA,
            pltpu.SemaphoreType.DMA,
            pltpu.SemaphoreType.DMA,
            pltpu.SemaphoreType.DMA,
        ],
        compiler_params=_sc_params(),
    )
    def edge_kernel(p_hbm, q_hbm, src_hbm, dst_hbm, ea0_hbm, ea1_hbm,
                    uw_hbm, out_hbm,
                    pa, pb, qa, qb, isrc_a, isrc_b, idst_a, idst_b,
                    ea0_v, ea1_v, out_v, uw_v,
                    sp_a, sp_b, sq_a, sq_b):
        c = lax.axis_index("c")
        s = lax.axis_index("s")
        wid = c * NS + s
        base = wid * epw
        pltpu.sync_copy(ea0_hbm.at[pl.ds(base, epw)], ea0_v)
        pltpu.sync_copy(ea1_hbm.at[pl.ds(base, epw)], ea1_v)
        pltpu.sync_copy(uw_hbm, uw_v)
        lane = lax.iota(jnp.int32, 16)

        def start(k, isrc, idst, p_v, q_v, sp, sq):
            pltpu.sync_copy(src_hbm.at[pl.ds(base + k * C, C)], isrc)
            pltpu.sync_copy(dst_hbm.at[pl.ds(base + k * C, C)], idst)
            pltpu.async_copy(p_hbm.at[isrc], p_v, sp)
            pltpu.async_copy(q_hbm.at[idst], q_v, sq)

        def finish(k, isrc, idst, p_v, q_v, sp, sq):
            pltpu.make_async_copy(p_hbm.at[isrc], p_v, sp).wait()
            pltpu.make_async_copy(q_hbm.at[idst], q_v, sq).wait()

            def grp(j2, carry):
                off = k * C + j2 * 16
                e0v = ea0_v[pl.ds(off, 16)]
                e1v = ea1_v[pl.ds(off, 16)]
                acc0 = uw_v[3, pl.ds(0, 16)]
                res = jnp.zeros((16,), jnp.float32)
                for e in range(16):
                    j = j2 * 16 + e
                    s0 = e0v[e]
                    s1 = e1v[e]
                    acc = acc0
                    for g in range(G):
                        sl = pl.ds(g * 16, 16)
                        hid = (p_v[j, sl] + q_v[j, sl]
                               + s0 * uw_v[0, sl] + s1 * uw_v[1, sl])
                        hid = jnp.maximum(hid, 0.0)
                        acc = acc + hid * uw_v[2, sl]
                    tot = jnp.sum(acc)
                    res = jnp.where(lane == e, tot, res)
                out_v[pl.ds(off, 16)] = res
                return carry

            lax.fori_loop(0, C // 16, grp, 0)

        start(0, isrc_a, idst_a, pa, qa, sp_a, sq_a)

        def body(i, carry):
            start(2 * i + 1, isrc_b, idst_b, pb, qb, sp_b, sq_b)
            finish(2 * i, isrc_a, idst_a, pa, qa, sp_a, sq_a)
            start(2 * i + 2, isrc_a, idst_a, pa, qa, sp_a, sq_a)
            finish(2 * i + 1, isrc_b, idst_b, pb, qb, sp_b, sq_b)
            return carry

        lax.fori_loop(0, (nchunks - 1) // 2, body, 0)
        finish(nchunks - 1, isrc_a, idst_a, pa, qa, sp_a, sq_a)
        pltpu.sync_copy(out_v, out_hbm.at[pl.ds(base, epw)])

    return edge_kernel


def _make_edge_gather_kernel(N, E, C):
    epw = E // (NC * NS)
    nchunks = epw // C

    assert nchunks % 2 == 1

    @functools.partial(
        pl.kernel,
        out_type=jax.ShapeDtypeStruct((E, H), jnp.float32),
        mesh=_sc_mesh(),
        scratch_types=[
            pltpu.VMEM((C, H), jnp.float32),
            pltpu.VMEM((C, H), jnp.float32),
            pltpu.VMEM((C, H), jnp.float32),
            pltpu.VMEM((C, H), jnp.float32),
            pltpu.VMEM((C,), jnp.int32),
            pltpu.VMEM((C,), jnp.int32),
            pltpu.VMEM((C,), jnp.int32),
            pltpu.VMEM((C,), jnp.int32),
            pltpu.SemaphoreType.DMA,
            pltpu.SemaphoreType.DMA,
            pltpu.SemaphoreType.DMA,
            pltpu.SemaphoreType.DMA,
        ],
    )
    def edge_kernel(p_hbm, q_hbm, src_hbm, dst_hbm, out_hbm,
                    pa, pb, qa, qb, isrc_a, isrc_b, idst_a, idst_b,
                    sp_a, sp_b, sq_a, sq_b):
        c = lax.axis_index("c")
        s = lax.axis_index("s")
        wid = c * NS + s
        base = wid * epw

        def start(k, isrc, idst, p_v, q_v, sp, sq):
            pltpu.sync_copy(src_hbm.at[pl.ds(base + k * C, C)], isrc)
            pltpu.sync_copy(dst_hbm.at[pl.ds(base + k * C, C)], idst)
            pltpu.async_copy(p_hbm.at[isrc], p_v, sp)
            pltpu.async_copy(q_hbm.at[idst], q_v, sq)

        def finish(k, isrc, idst, p_v, q_v, sp, sq):
            pltpu.make_async_copy(p_hbm.at[isrc], p_v, sp).wait()
            pltpu.make_async_copy(q_hbm.at[idst], q_v, sq).wait()

            def add_body(j, acc):
                for g in range(H // 16):
                    sl = pl.ds(g * 16, 16)
                    p_v[j, sl] = p_v[j, sl] + q_v[j, sl]
                return acc

            lax.fori_loop(0, C, add_body, 0)
            pltpu.sync_copy(p_v, out_hbm.at[pl.ds(base + k * C, C)])

        start(0, isrc_a, idst_a, pa, qa, sp_a, sq_a)

        def body(i, carry):
            start(2 * i + 1, isrc_b, idst_b, pb, qb, sp_b, sq_b)
            finish(2 * i, isrc_a, idst_a, pa, qa, sp_a, sq_a)
            start(2 * i + 2, isrc_a, idst_a, pa, qa, sp_a, sq_a)
            finish(2 * i + 1, isrc_b, idst_b, pb, qb, sp_b, sq_b)
            return carry

        lax.fori_loop(0, (nchunks - 1) // 2, body, 0)
        finish(nchunks - 1, isrc_a, idst_a, pa, qa, sp_a, sq_a)

    return edge_kernel


def _tc_pre(x, W_gcn, h_prev, W_hhT, b_hh):
    N = x.shape[0]
    R = 1000
    grid = (N // R,)

    def body(x_ref, wg_ref, hp_ref, whh_ref, bhh_ref, xw_ref, gh_ref):
        xw_ref[...] = jnp.dot(x_ref[...], wg_ref[...],
                              preferred_element_type=jnp.float32)
        gh_ref[...] = (jnp.dot(hp_ref[...], whh_ref[...],
                               preferred_element_type=jnp.float32)
                       + bhh_ref[...][None, :])

    return pl.pallas_call(
        body,
        grid=grid,
        in_specs=[
            pl.BlockSpec((R, H), lambda i: (i, 0)),
            pl.BlockSpec((H, H), lambda i: (0, 0)),
            pl.BlockSpec((R, H), lambda i: (i, 0)),
            pl.BlockSpec((H, 3 * H), lambda i: (0, 0)),
            pl.BlockSpec((3 * H,), lambda i: (0,)),
        ],
        out_specs=[
            pl.BlockSpec((R, H), lambda i: (i, 0)),
            pl.BlockSpec((R, 3 * H), lambda i: (i, 0)),
        ],
        out_shape=[
            jax.ShapeDtypeStruct((N, H), jnp.float32),
            jax.ShapeDtypeStruct((N, 3 * H), jnp.float32),
        ],
    )(x, W_gcn, h_prev, W_hhT, b_hh)


def _tc_scale(xw, deg2):
    N = xw.shape[0]
    R = 1000
    grid = (N // R,)

    def body(xw_ref, deg_ref, y_ref):
        deg = deg_ref[0, :, 0] + deg_ref[1, :, 0] + 1.0
        y_ref[...] = xw_ref[...] * lax.rsqrt(deg)[:, None]

    return pl.pallas_call(
        body,
        grid=grid,
        in_specs=[
            pl.BlockSpec((R, H), lambda i: (i, 0)),
            pl.BlockSpec((NC, R, H), lambda i: (0, i, 0)),
        ],
        out_specs=pl.BlockSpec((R, H), lambda i: (i, 0)),
        out_shape=jax.ShapeDtypeStruct((N, H), jnp.float32),
    )(xw, deg2)


def _tc_mid(S2, y, deg2, b_gcn, W_ihT, b_ih, gh, h_prev, W1aT, W1bT, b1):
    N = y.shape[0]
    R = 1000
    grid = (N // R,)

    def body(s2_ref, y_ref, deg_ref, bg_ref, wih_ref, bih_ref, gh_ref,
             hp_ref, w1a_ref, w1b_ref, b1_ref, hn_ref, p_ref, q_ref):
        deg = deg_ref[0, :, 0] + deg_ref[1, :, 0] + 1.0
        dinv = lax.rsqrt(deg)
        ssum = s2_ref[0] + s2_ref[1] + y_ref[...]
        agg = ssum * dinv[:, None] + bg_ref[...][None, :]
        h_curr = jnp.maximum(agg, 0.0)
        gi = (jnp.dot(h_curr, wih_ref[...], preferred_element_type=jnp.float32)
              + bih_ref[...][None, :])
        gh = gh_ref[...]
        r = jax.nn.sigmoid(gi[:, :H] + gh[:, :H])
        z = jax.nn.sigmoid(gi[:, H:2 * H] + gh[:, H:2 * H])
        ng = jnp.tanh(gi[:, 2 * H:] + r * gh[:, 2 * H:])
        hp = hp_ref[...]
        hn = (1.0 - z) * ng + z * hp
        hn_ref[...] = hn
        p_ref[...] = (jnp.dot(hn, w1a_ref[...], preferred_element_type=jnp.float32)
                      + b1_ref[...][None, :])
        q_ref[...] = jnp.dot(hn, w1b_ref[...], preferred_element_type=jnp.float32)

    return pl.pallas_call(
        body,
        grid=grid,
        in_specs=[
            pl.BlockSpec((NC, R, H), lambda i: (0, i, 0)),
            pl.BlockSpec((R, H), lambda i: (i, 0)),
            pl.BlockSpec((NC, R, H), lambda i: (0, i, 0)),
            pl.BlockSpec((H,), lambda i: (0,)),
            pl.BlockSpec((H, 3 * H), lambda i: (0, 0)),
            pl.BlockSpec((3 * H,), lambda i: (0,)),
            pl.BlockSpec((R, 3 * H), lambda i: (i, 0)),
            pl.BlockSpec((R, H), lambda i: (i, 0)),
            pl.BlockSpec((H, H), lambda i: (0, 0)),
            pl.BlockSpec((H, H), lambda i: (0, 0)),
            pl.BlockSpec((H,), lambda i: (0,)),
        ],
        out_specs=[
            pl.BlockSpec((R, H), lambda i: (i, 0)),
            pl.BlockSpec((R, H), lambda i: (i, 0)),
            pl.BlockSpec((R, H), lambda i: (i, 0)),
        ],
        out_shape=[
            jax.ShapeDtypeStruct((N, H), jnp.float32),
            jax.ShapeDtypeStruct((N, H), jnp.float32),
            jax.ShapeDtypeStruct((N, H), jnp.float32),
        ],
    )(S2, y, deg2, b_gcn, W_ihT, b_ih, gh, h_prev, W1aT, W1bT, b1)


def _tc_edge_mlp(Rpq, ea, u01, w2m, diag_m, b2):
    E = Rpq.shape[0]
    B = 512
    grid = (E // B,)

    def body(r_ref, ea_ref, u_ref, w2_ref, m_ref, b2_ref, out_ref):
        eau = jnp.dot(ea_ref[...], u_ref[...], preferred_element_type=jnp.float32)
        hid = jnp.maximum(r_ref[...] + eau, 0.0)
        t = jnp.dot(hid, w2_ref[...], preferred_element_type=jnp.float32)
        v = jnp.sum(t.reshape(B // 64, 64, H) * m_ref[...][None], axis=1)
        out_ref[...] = v[:, :64] + b2_ref[...][0]

    out2d = pl.pallas_call(
        body,
        grid=grid,
        in_specs=[
            pl.BlockSpec((B, H), lambda i: (i, 0)),
            pl.BlockSpec((B, 2), lambda i: (i, 0)),
            pl.BlockSpec((2, H), lambda i: (0, 0)),
            pl.BlockSpec((H, H), lambda i: (0, 0)),
            pl.BlockSpec((64, H), lambda i: (0, 0)),
            pl.BlockSpec((1,), lambda i: (0,)),
        ],
        out_specs=pl.BlockSpec((B // 64, 64), lambda i: (i, 0)),
        out_shape=jax.ShapeDtypeStruct((E // 64, 64), jnp.float32),
    )(Rpq, ea, u01, w2m, diag_m, b2)
    return out2d.reshape(E)


def kernel(x, edge_index, edge_attr, h_prev, W_gcn, b_gcn, W_ih, W_hh,
           b_ih, b_hh, W1, b1, W2, b2):
    N = x.shape[0]
    E = edge_index.shape[1]
    C = 80
    Np = ((N + NS * 8 - 1) // (NS * 8)) * (NS * 8)

    src = edge_index[0]
    dst = edge_index[1]
    onesH = jnp.ones((C, H), jnp.float32)
    zerosH = jnp.zeros((Np // NS, H), jnp.float32)

    deg2 = _make_deg_kernel(Np, E, C)(dst, onesH, zerosH)
    xw, gh = _tc_pre(x, W_gcn, h_prev, W_hh.T, b_hh)
    y = _tc_scale(xw, deg2)
    S2 = _make_scatter_kernel(Np, E, C)(y, src, dst, zerosH)
    h_next, P, Q = _tc_mid(S2, y, deg2, b_gcn, W_ih.T, b_ih, gh, h_prev,
                           W1[:, :H].T, W1[:, H:2 * H].T, b1)
    uw = jnp.stack([W1[:, 2 * H], W1[:, 2 * H + 1], W2[0],
                    jnp.full((H,), b2[0] / 16.0, jnp.float32)])
    out = _make_edge_mlp_kernel(N, E, C)(P, Q, src, dst,
                                         edge_attr[:, 0], edge_attr[:, 1], uw)
    return (out, h_next)
